# baseline (speedup 1.0000x reference)
import functools

import jax
import jax.numpy as jnp
from jax import lax
from jax.experimental import pallas as pl
from jax.experimental.pallas import tpu as pltpu
from jax.experimental.pallas import tpu_sc as plsc

N = 10000
D = 128
H = 128
E = 320000
EP = 10000

NC = 2
NS = 16
NW = NC * NS


KA = 80
EDGES_PER_TILE = E // NW
NGROUPS = EDGES_PER_TILE // KA
NWCH = N // KA
NRING = 3


def _sc_aggregate_body(h_hbm, src_hbm, dst_hbm, agg_out, deg_out,
                       src0, dst0, src1, dst1, src2, dst2,
                       ones_v, rows0, rows1, rows2, zdeg_v,
                       agg_sh, deg_sh,
                       sem_g, sem_s, sem_w, sem_i0, sem_i1, sem_i2):
    c = lax.axis_index("c")
    s = lax.axis_index("s")
    rows = [rows0, rows1, rows2]
    isrc = [src0, src1, src2]
    idst = [dst0, dst1, dst2]
    isem = [sem_i0, sem_i1, sem_i2]

    gbase = (c * NS + s) * NGROUPS
    pltpu.async_copy(src_hbm.at[gbase], isrc[0], isem[0])
    pltpu.async_copy(dst_hbm.at[gbase], idst[0], isem[0])

    zero16 = jnp.zeros((16,), jnp.float32)
    one16 = jnp.ones((16,), jnp.float32)
    for j in range(KA // 16):
        ones_v[pl.ds(j * 16, 16)] = one16
    for r in range(KA):
        for q in range(H // 16):
            rows0[r, pl.ds(q * 16, 16)] = zero16
    for j in range(2000 // 16):
        zdeg_v[pl.ds(j * 16, 16)] = zero16

    nz = (NWCH + NS - 1) // NS

    def zfire(k, carry):
        cid = s + k * NS

        @pl.when(cid < NWCH)
        def _():
            pltpu.async_copy(rows0, agg_sh.at[pl.ds(cid * KA, KA)], sem_s)

        return carry

    def zdrain(k, carry):
        cid = s + k * NS

        @pl.when(cid < NWCH)
        def _():
            pltpu.make_async_copy(
                rows0, agg_sh.at[pl.ds(cid * KA, KA)], sem_s).wait()

        return carry

    lax.fori_loop(0, nz, zfire, 0)

    @pl.when(s < 5)
    def _zero_deg():
        pltpu.async_copy(zdeg_v, deg_sh.at[pl.ds(s * 2000, 2000)], sem_s)

    lax.fori_loop(0, nz, zdrain, 0)

    @pl.when(s < 5)
    def _zero_deg_wait():
        pltpu.make_async_copy(
            zdeg_v, deg_sh.at[pl.ds(s * 2000, 2000)], sem_s).wait()

    plsc.subcore_barrier()

    def process(g, j, do_drain, do_prefetch):
        pj = (j + 1) % NRING
        if do_drain:
            pltpu.make_async_copy(
                rows[pj], agg_sh.at[idst[pj].at[0]], sem_s).wait()
            pltpu.make_async_copy(
                ones_v, deg_sh.at[idst[pj].at[0]], sem_s).wait()
        if do_prefetch:
            pltpu.async_copy(src_hbm.at[gbase + g + 1], isrc[pj], isem[pj])
            pltpu.async_copy(dst_hbm.at[gbase + g + 1], idst[pj], isem[pj])
        pltpu.make_async_copy(src_hbm.at[gbase + g], isrc[j], isem[j]).wait()
        pltpu.make_async_copy(dst_hbm.at[gbase + g], idst[j], isem[j]).wait()
        pltpu.async_copy(h_hbm.at[isrc[j].at[0]], rows[j], sem_g)
        pltpu.make_async_copy(h_hbm.at[isrc[j].at[0]], rows[j], sem_g).wait()
        pltpu.async_copy(rows[j], agg_sh.at[idst[j].at[0]], sem_s, add=True)
        pltpu.async_copy(ones_v, deg_sh.at[idst[j].at[0]], sem_s, add=True)

    process(0, 0, False, True)
    process(1, 1, False, True)
    process(2, 2, True, True)

    def triple(p, carry):
        g = 3 + 3 * p
        process(g, 0, True, True)
        process(g + 1, 1, True, True)
        process(g + 2, 2, True, True)
        return carry

    lax.fori_loop(0, (NGROUPS - 5) // 3, triple, 0)
    process(NGROUPS - 2, 0, True, True)
    process(NGROUPS - 1, 1, True, False)

    for j in (0, 1):
        pltpu.make_async_copy(
            rows[j], agg_sh.at[idst[j].at[0]], sem_s).wait()
        pltpu.make_async_copy(
            ones_v, deg_sh.at[idst[j].at[0]], sem_s).wait()

    plsc.subcore_barrier()

    nw_rounds = (NWCH + NS * NRING - 1) // (NS * NRING)

    def wround(r, carry):
        for b in range(NRING):
            cid = s + (r * NRING + b) * NS

            @pl.when(cid < NWCH)
            def _():
                pltpu.async_copy(agg_sh.at[pl.ds(cid * KA, KA)],
                                 rows[b], sem_g)

        for b in range(NRING):
            cid = s + (r * NRING + b) * NS

            @pl.when(cid < NWCH)
            def _():
                pltpu.make_async_copy(agg_sh.at[pl.ds(cid * KA, KA)],
                                      rows[b], sem_g).wait()
                pltpu.async_copy(rows[b],
                                 agg_out.at[c, pl.ds(cid * KA, KA)],
                                 sem_w)

        for b in range(NRING):
            cid = s + (r * NRING + b) * NS

            @pl.when(cid < NWCH)
            def _():
                pltpu.make_async_copy(
                    rows[b], agg_out.at[c, pl.ds(cid * KA, KA)],
                    sem_w).wait()

        return carry

    lax.fori_loop(0, nw_rounds, wround, 0)

    @pl.when(s < 5)
    def _write_deg():
        pltpu.sync_copy(deg_sh.at[pl.ds(s * 2000, 2000)], zdeg_v)
        pltpu.sync_copy(zdeg_v, deg_out.at[pl.ds(c * N + s * 2000, 2000)])


def _sc_aggregate(h, src_arr, dst_arr):
    mesh = plsc.VectorSubcoreMesh(core_axis_name="c", subcore_axis_name="s")
    src3 = src_arr.reshape(NW * NGROUPS, 1, KA)
    dst3 = dst_arr.reshape(NW * NGROUPS, 1, KA)
    return pl.kernel(
        _sc_aggregate_body,
        out_type=[jax.ShapeDtypeStruct((NC, N, H), jnp.float32),
                  jax.ShapeDtypeStruct((NC * N,), jnp.float32)],
        mesh=mesh,
        scratch_types=[
            pltpu.VMEM((1, KA), jnp.int32),
            pltpu.VMEM((1, KA), jnp.int32),
            pltpu.VMEM((1, KA), jnp.int32),
            pltpu.VMEM((1, KA), jnp.int32),
            pltpu.VMEM((1, KA), jnp.int32),
            pltpu.VMEM((1, KA), jnp.int32),
            pltpu.VMEM((KA,), jnp.float32),
            pltpu.VMEM((KA, H), jnp.float32),
            pltpu.VMEM((KA, H), jnp.float32),
            pltpu.VMEM((KA, H), jnp.float32),
            pltpu.VMEM((2000,), jnp.float32),
            pltpu.VMEM_SHARED((N, H), jnp.float32),
            pltpu.VMEM_SHARED((N,), jnp.float32),
            pltpu.SemaphoreType.DMA,
            pltpu.SemaphoreType.DMA,
            pltpu.SemaphoreType.DMA,
            pltpu.SemaphoreType.DMA,
            pltpu.SemaphoreType.DMA,
            pltpu.SemaphoreType.DMA,
        ],
    )(h, src3, dst3)


K = 80
NCHUNKS_SCORE = 2 * EP // K
CHUNKS_PER_ARRAY = EP // K


def _sc_scores_body(h_hbm, comb_hbm, prows_out,
                    aidx_v, bidx_v, rows_a, rows_b, sem_a, sem_b):
    c = lax.axis_index("c")
    s = lax.axis_index("s")
    w = s * NC + c

    def do_chunk(cid):
        g = cid // CHUNKS_PER_ARRAY
        off = (cid % CHUNKS_PER_ARRAY) * K
        pltpu.sync_copy(comb_hbm.at[pl.ds(2 * g * EP + off, K)], aidx_v)
        pltpu.sync_copy(comb_hbm.at[pl.ds((2 * g + 1) * EP + off, K)], bidx_v)
        cp_a = pltpu.async_copy(h_hbm.at[aidx_v], rows_a, sem_a)
        cp_b = pltpu.async_copy(h_hbm.at[bidx_v], rows_b, sem_b)
        cp_a.wait()
        cp_b.wait()

        def ebody(e, carry):
            for q in range(H // 16):
                sl = pl.ds(q * 16, 16)
                rows_a[e, sl] = rows_a[e, sl] * rows_b[e, sl]
            return carry

        lax.fori_loop(0, K, ebody, 0)
        pltpu.sync_copy(rows_a, prows_out.at[pl.ds(g * EP + off, K)])

    def loop(k, carry):
        cid = w + k * NW

        @pl.when(cid < NCHUNKS_SCORE)
        def _():
            do_chunk(cid)

        return carry

    nmax = (NCHUNKS_SCORE + NW - 1) // NW
    lax.fori_loop(0, nmax, loop, 0)


def _sc_scores(h, comb):
    mesh = plsc.VectorSubcoreMesh(core_axis_name="c", subcore_axis_name="s")
    return pl.kernel(
        _sc_scores_body,
        out_type=jax.ShapeDtypeStruct((2 * EP, H), jnp.float32),
        mesh=mesh,
        scratch_types=[
            pltpu.VMEM((K,), jnp.int32),
            pltpu.VMEM((K,), jnp.int32),
            pltpu.VMEM((K, H), jnp.float32),
            pltpu.VMEM((K, H), jnp.float32),
            pltpu.SemaphoreType.DMA,
            pltpu.SemaphoreType.DMA,
        ],
    )(h, comb)


def _dots_body(p_ref, out_ref):
    out_ref[...] = jnp.sum(p_ref[...], axis=-1, keepdims=True)


def _dots_tc(prows):
    bn = 2000
    return pl.pallas_call(
        _dots_body,
        grid=(2 * EP // bn,),
        in_specs=[pl.BlockSpec((bn, H), lambda i: (i, 0))],
        out_specs=pl.BlockSpec((bn, 1), lambda i: (i, 0)),
        out_shape=jax.ShapeDtypeStruct((2 * EP, 1), jnp.float32),
    )(prows)


def _ln(t, g, b):
    m = jnp.mean(t, axis=-1, keepdims=True)
    v = jnp.mean((t - m) * (t - m), axis=-1, keepdims=True)
    return (t - m) * lax.rsqrt(v + 1e-5) * g + b


def _proj_body(x_ref, Win_ref, bin_ref, lA_ref, lB_ref, g_ref, b_ref, out_ref):
    x = x_ref[...]
    base = jnp.dot(x, Win_ref[...], preferred_element_type=jnp.float32)
    ada = jnp.dot(jnp.dot(x, lA_ref[...], preferred_element_type=jnp.float32),
                  lB_ref[...], preferred_element_type=jnp.float32)
    t = base + ada + bin_ref[...]
    out_ref[...] = _ln(t, g_ref[...], b_ref[...])


_BN = 1000


def _proj_tc(x, W_in, b_in, lora_A, lora_B, pn_g, pn_b):
    full = lambda shape: pl.BlockSpec(shape, lambda i: tuple(0 for _ in shape))
    return pl.pallas_call(
        _proj_body,
        grid=(N // _BN,),
        in_specs=[pl.BlockSpec((_BN, D), lambda i: (i, 0)),
                  full((D, H)), full((1, H)), full((D, 16)), full((16, H)),
                  full((1, H)), full((1, H))],
        out_specs=pl.BlockSpec((_BN, H), lambda i: (i, 0)),
        out_shape=jax.ShapeDtypeStruct((N, H), jnp.float32),
    )(x, W_in, b_in.reshape(1, H), lora_A, lora_B,
      pn_g.reshape(1, H), pn_b.reshape(1, H))


def _layer_body(residual, h_ref, agg_ref, deg_ref, Ws_ref, Wn_ref, bb_ref,
                g_ref, b_ref, out_ref):
    h = h_ref[...]
    agg = agg_ref[0] + agg_ref[1]
    deg = deg_ref[0] + deg_ref[1]
    agg = agg / jnp.maximum(deg, 1.0)
    t = (jnp.dot(h, Ws_ref[...], preferred_element_type=jnp.float32)
         + jnp.dot(agg, Wn_ref[...], preferred_element_type=jnp.float32)
         + bb_ref[...])
    t = jnp.maximum(t, 0.0)
    if residual:
        t = t + h
    out_ref[...] = _ln(t, g_ref[...], b_ref[...])


def _layer_tc(h, aggp, degp, Ws, Wn, bb, ln_g, ln_b, residual):
    full = lambda shape: pl.BlockSpec(shape, lambda i: tuple(0 for _ in shape))
    return pl.pallas_call(
        functools.partial(_layer_body, residual),
        grid=(N // _BN,),
        in_specs=[pl.BlockSpec((_BN, H), lambda i: (i, 0)),
                  pl.BlockSpec((NC, _BN, H), lambda i: (0, i, 0)),
                  pl.BlockSpec((NC, _BN, 1), lambda i: (0, i, 0)),
                  full((H, H)), full((H, H)), full((1, H)),
                  full((1, H)), full((1, H))],
        out_specs=pl.BlockSpec((_BN, H), lambda i: (i, 0)),
        out_shape=jax.ShapeDtypeStruct((N, H), jnp.float32),
    )(h, aggp, degp.reshape(NC, N, 1), Ws, Wn, bb.reshape(1, H),
      ln_g.reshape(1, H), ln_b.reshape(1, H))


def kernel(x, edge_index_l0, edge_index_l1, pos_edge_index, neg_edge_index,
           W_in, b_in, lora_A, lora_B, pn_g, pn_b,
           Ws0, Wn0, bb0, ln0_g, ln0_b,
           Ws1, Wn1, bb1, ln1_g, ln1_b):
    h0 = _proj_tc(x, W_in, b_in, lora_A, lora_B, pn_g, pn_b)
    aggp0, degp0 = _sc_aggregate(h0, edge_index_l0[0], edge_index_l0[1])
    h1 = _layer_tc(h0, aggp0, degp0.reshape(NC, N, 1),
                   Ws0, Wn0, bb0, ln0_g, ln0_b, residual=False)
    aggp1, degp1 = _sc_aggregate(h1, edge_index_l1[0], edge_index_l1[1])
    h2 = _layer_tc(h1, aggp1, degp1.reshape(NC, N, 1),
                   Ws1, Wn1, bb1, ln1_g, ln1_b, residual=True)
    comb = jnp.concatenate(
        [pos_edge_index.reshape(-1), neg_edge_index.reshape(-1)])
    prows = _sc_scores(h2, comb)
    scores = _dots_tc(prows)[:, 0]
    return (scores[:EP], scores[EP:])


# speedup vs baseline: 1.3226x; 1.3226x over previous
import functools

import jax
import jax.numpy as jnp
from jax import lax
from jax.experimental import pallas as pl
from jax.experimental.pallas import tpu as pltpu
from jax.experimental.pallas import tpu_sc as plsc

N = 10000
D = 128
H = 128
E = 320000
EP = 10000

NC = 2
NS = 16
NW = NC * NS


KA = 80
EDGES_PER_TILE = E // NW
NGROUPS = EDGES_PER_TILE // KA
NWCH = N // KA
NRING = 4


def _sc_aggregate_body(h_hbm, src_hbm, dst_hbm, agg_out, deg_out,
                       src0, dst0, src1, dst1, src2, dst2, src3, dst3,
                       ones_v, rows0, rows1, rows2, rows3, zdeg_v,
                       agg_sh, deg_sh,
                       sem_g, sem_s, sem_w, sem_i0, sem_i1, sem_i2, sem_i3):
    c = lax.axis_index("c")
    s = lax.axis_index("s")
    rows = [rows0, rows1, rows2, rows3]
    isrc = [src0, src1, src2, src3]
    idst = [dst0, dst1, dst2, dst3]
    isem = [sem_i0, sem_i1, sem_i2, sem_i3]

    gbase = (c * NS + s) * NGROUPS
    pltpu.async_copy(src_hbm.at[gbase], isrc[0], isem[0])
    pltpu.async_copy(dst_hbm.at[gbase], idst[0], isem[0])
    pltpu.async_copy(src_hbm.at[gbase + 1], isrc[1], isem[1])
    pltpu.async_copy(dst_hbm.at[gbase + 1], idst[1], isem[1])

    zero16 = jnp.zeros((16,), jnp.float32)
    one16 = jnp.ones((16,), jnp.float32)
    for j in range(KA // 16):
        ones_v[pl.ds(j * 16, 16)] = one16
    for r in range(KA):
        for q in range(H // 16):
            rows3[r, pl.ds(q * 16, 16)] = zero16
    for j in range(2000 // 16):
        zdeg_v[pl.ds(j * 16, 16)] = zero16

    nz = (NWCH + NS - 1) // NS

    def zfire(k, carry):
        cid = s + k * NS

        @pl.when(cid < NWCH)
        def _():
            pltpu.async_copy(rows3, agg_sh.at[pl.ds(cid * KA, KA)], sem_s)

        return carry

    def zdrain(k, carry):
        cid = s + k * NS

        @pl.when(cid < NWCH)
        def _():
            pltpu.make_async_copy(
                rows3, agg_sh.at[pl.ds(cid * KA, KA)], sem_s).wait()

        return carry

    lax.fori_loop(0, nz, zfire, 0)

    @pl.when(s < 5)
    def _zero_deg():
        pltpu.async_copy(zdeg_v, deg_sh.at[pl.ds(s * 2000, 2000)], sem_s)

    pltpu.make_async_copy(src_hbm.at[gbase], isrc[0], isem[0]).wait()
    pltpu.make_async_copy(dst_hbm.at[gbase], idst[0], isem[0]).wait()
    pltpu.async_copy(h_hbm.at[isrc[0].at[0]], rows[0], sem_g)

    lax.fori_loop(0, nz, zdrain, 0)

    @pl.when(s < 5)
    def _zero_deg_wait():
        pltpu.make_async_copy(
            zdeg_v, deg_sh.at[pl.ds(s * 2000, 2000)], sem_s).wait()

    plsc.subcore_barrier()

    def process(g, j, do_drain, do_prefetch, do_gather_next):
        nj = (j + 1) % NRING
        pj = (j + 2) % NRING
        if do_drain:
            pltpu.make_async_copy(
                rows[pj], agg_sh.at[idst[pj].at[0]], sem_s).wait()
            pltpu.make_async_copy(
                ones_v, deg_sh.at[idst[pj].at[0]], sem_s).wait()
        if do_prefetch:
            pltpu.async_copy(src_hbm.at[gbase + g + 2], isrc[pj], isem[pj])
            pltpu.async_copy(dst_hbm.at[gbase + g + 2], idst[pj], isem[pj])
        if do_gather_next:
            pltpu.make_async_copy(
                src_hbm.at[gbase + g + 1], isrc[nj], isem[nj]).wait()
            pltpu.make_async_copy(
                dst_hbm.at[gbase + g + 1], idst[nj], isem[nj]).wait()
            pltpu.async_copy(h_hbm.at[isrc[nj].at[0]], rows[nj], sem_g)
        pltpu.make_async_copy(h_hbm.at[isrc[j].at[0]], rows[j], sem_g).wait()
        pltpu.async_copy(rows[j], agg_sh.at[idst[j].at[0]], sem_s, add=True)
        pltpu.async_copy(ones_v, deg_sh.at[idst[j].at[0]], sem_s, add=True)

    process(0, 0, False, True, True)
    process(1, 1, False, True, True)

    def quad(p, carry):
        g = 2 + 4 * p
        process(g, 2, True, True, True)
        process(g + 1, 3, True, True, True)
        process(g + 2, 0, True, True, True)
        process(g + 3, 1, True, True, True)
        return carry

    lax.fori_loop(0, (NGROUPS - 5) // 4, quad, 0)
    process(NGROUPS - 3, 2, True, True, True)
    process(NGROUPS - 2, 3, True, False, True)
    process(NGROUPS - 1, 0, True, False, False)

    for j in (3, 0):
        pltpu.make_async_copy(
            rows[j], agg_sh.at[idst[j].at[0]], sem_s).wait()
        pltpu.make_async_copy(
            ones_v, deg_sh.at[idst[j].at[0]], sem_s).wait()

    plsc.subcore_barrier()

    nw_rounds = (NWCH + NS * NRING - 1) // (NS * NRING)

    def wround(r, carry):
        for b in range(NRING):
            cid = s + (r * NRING + b) * NS

            @pl.when(cid < NWCH)
            def _():
                pltpu.async_copy(agg_sh.at[pl.ds(cid * KA, KA)],
                                 rows[b], sem_g)

        for b in range(NRING):
            cid = s + (r * NRING + b) * NS

            @pl.when(cid < NWCH)
            def _():
                pltpu.make_async_copy(agg_sh.at[pl.ds(cid * KA, KA)],
                                      rows[b], sem_g).wait()
                pltpu.async_copy(rows[b],
                                 agg_out.at[c, pl.ds(cid * KA, KA)],
                                 sem_w)

        for b in range(NRING):
            cid = s + (r * NRING + b) * NS

            @pl.when(cid < NWCH)
            def _():
                pltpu.make_async_copy(
                    rows[b], agg_out.at[c, pl.ds(cid * KA, KA)],
                    sem_w).wait()

        return carry

    lax.fori_loop(0, nw_rounds, wround, 0)

    @pl.when(s < 5)
    def _write_deg():
        pltpu.sync_copy(deg_sh.at[pl.ds(s * 2000, 2000)], zdeg_v)
        pltpu.sync_copy(zdeg_v, deg_out.at[pl.ds(c * N + s * 2000, 2000)])


def _sc_aggregate(h, src_arr, dst_arr):
    mesh = plsc.VectorSubcoreMesh(core_axis_name="c", subcore_axis_name="s")
    src3 = src_arr.reshape(NW * NGROUPS, 1, KA)
    dst3 = dst_arr.reshape(NW * NGROUPS, 1, KA)
    return pl.kernel(
        _sc_aggregate_body,
        out_type=[jax.ShapeDtypeStruct((NC, N, H), jnp.float32),
                  jax.ShapeDtypeStruct((NC * N,), jnp.float32)],
        mesh=mesh,
        scratch_types=[
            pltpu.VMEM((1, KA), jnp.int32),
            pltpu.VMEM((1, KA), jnp.int32),
            pltpu.VMEM((1, KA), jnp.int32),
            pltpu.VMEM((1, KA), jnp.int32),
            pltpu.VMEM((1, KA), jnp.int32),
            pltpu.VMEM((1, KA), jnp.int32),
            pltpu.VMEM((1, KA), jnp.int32),
            pltpu.VMEM((1, KA), jnp.int32),
            pltpu.VMEM((KA,), jnp.float32),
            pltpu.VMEM((KA, H), jnp.float32),
            pltpu.VMEM((KA, H), jnp.float32),
            pltpu.VMEM((KA, H), jnp.float32),
            pltpu.VMEM((KA, H), jnp.float32),
            pltpu.VMEM((2000,), jnp.float32),
            pltpu.VMEM_SHARED((N, H), jnp.float32),
            pltpu.VMEM_SHARED((N,), jnp.float32),
            pltpu.SemaphoreType.DMA,
            pltpu.SemaphoreType.DMA,
            pltpu.SemaphoreType.DMA,
            pltpu.SemaphoreType.DMA,
            pltpu.SemaphoreType.DMA,
            pltpu.SemaphoreType.DMA,
            pltpu.SemaphoreType.DMA,
        ],
    )(h, src3, dst3)


K = 80
NCHUNKS_SCORE = 2 * EP // K
CHUNKS_PER_ARRAY = EP // K


def _sc_scores_body(h_hbm, comb_hbm, prows_out,
                    aidx_v, bidx_v, rows_a, rows_b, sem_a, sem_b):
    c = lax.axis_index("c")
    s = lax.axis_index("s")
    w = s * NC + c

    def do_chunk(cid):
        g = cid // CHUNKS_PER_ARRAY
        off = (cid % CHUNKS_PER_ARRAY) * K
        pltpu.sync_copy(comb_hbm.at[pl.ds(2 * g * EP + off, K)], aidx_v)
        pltpu.sync_copy(comb_hbm.at[pl.ds((2 * g + 1) * EP + off, K)], bidx_v)
        cp_a = pltpu.async_copy(h_hbm.at[aidx_v], rows_a, sem_a)
        cp_b = pltpu.async_copy(h_hbm.at[bidx_v], rows_b, sem_b)
        cp_a.wait()
        cp_b.wait()

        def ebody(e, carry):
            for q in range(H // 16):
                sl = pl.ds(q * 16, 16)
                rows_a[e, sl] = rows_a[e, sl] * rows_b[e, sl]
            return carry

        lax.fori_loop(0, K, ebody, 0)
        pltpu.sync_copy(rows_a, prows_out.at[pl.ds(g * EP + off, K)])

    def loop(k, carry):
        cid = w + k * NW

        @pl.when(cid < NCHUNKS_SCORE)
        def _():
            do_chunk(cid)

        return carry

    nmax = (NCHUNKS_SCORE + NW - 1) // NW
    lax.fori_loop(0, nmax, loop, 0)


def _sc_scores(h, comb):
    mesh = plsc.VectorSubcoreMesh(core_axis_name="c", subcore_axis_name="s")
    return pl.kernel(
        _sc_scores_body,
        out_type=jax.ShapeDtypeStruct((2 * EP, H), jnp.float32),
        mesh=mesh,
        scratch_types=[
            pltpu.VMEM((K,), jnp.int32),
            pltpu.VMEM((K,), jnp.int32),
            pltpu.VMEM((K, H), jnp.float32),
            pltpu.VMEM((K, H), jnp.float32),
            pltpu.SemaphoreType.DMA,
            pltpu.SemaphoreType.DMA,
        ],
    )(h, comb)


def _dots_body(p_ref, out_ref):
    out_ref[...] = jnp.sum(p_ref[...], axis=-1, keepdims=True)


def _dots_tc(prows):
    bn = 2000
    return pl.pallas_call(
        _dots_body,
        grid=(2 * EP // bn,),
        in_specs=[pl.BlockSpec((bn, H), lambda i: (i, 0))],
        out_specs=pl.BlockSpec((bn, 1), lambda i: (i, 0)),
        out_shape=jax.ShapeDtypeStruct((2 * EP, 1), jnp.float32),
    )(prows)


def _ln(t, g, b):
    m = jnp.mean(t, axis=-1, keepdims=True)
    v = jnp.mean((t - m) * (t - m), axis=-1, keepdims=True)
    return (t - m) * lax.rsqrt(v + 1e-5) * g + b


def _proj_body(x_ref, Win_ref, bin_ref, lA_ref, lB_ref, g_ref, b_ref, out_ref):
    x = x_ref[...]
    base = jnp.dot(x, Win_ref[...], preferred_element_type=jnp.float32)
    ada = jnp.dot(jnp.dot(x, lA_ref[...], preferred_element_type=jnp.float32),
                  lB_ref[...], preferred_element_type=jnp.float32)
    t = base + ada + bin_ref[...]
    out_ref[...] = _ln(t, g_ref[...], b_ref[...])


_BN = 1000


def _proj_tc(x, W_in, b_in, lora_A, lora_B, pn_g, pn_b):
    full = lambda shape: pl.BlockSpec(shape, lambda i: tuple(0 for _ in shape))
    return pl.pallas_call(
        _proj_body,
        grid=(N // _BN,),
        in_specs=[pl.BlockSpec((_BN, D), lambda i: (i, 0)),
                  full((D, H)), full((1, H)), full((D, 16)), full((16, H)),
                  full((1, H)), full((1, H))],
        out_specs=pl.BlockSpec((_BN, H), lambda i: (i, 0)),
        out_shape=jax.ShapeDtypeStruct((N, H), jnp.float32),
    )(x, W_in, b_in.reshape(1, H), lora_A, lora_B,
      pn_g.reshape(1, H), pn_b.reshape(1, H))


def _layer_body(residual, h_ref, agg_ref, deg_ref, Ws_ref, Wn_ref, bb_ref,
                g_ref, b_ref, out_ref):
    h = h_ref[...]
    agg = agg_ref[0] + agg_ref[1]
    deg = deg_ref[0] + deg_ref[1]
    agg = agg / jnp.maximum(deg, 1.0)
    t = (jnp.dot(h, Ws_ref[...], preferred_element_type=jnp.float32)
         + jnp.dot(agg, Wn_ref[...], preferred_element_type=jnp.float32)
         + bb_ref[...])
    t = jnp.maximum(t, 0.0)
    if residual:
        t = t + h
    out_ref[...] = _ln(t, g_ref[...], b_ref[...])


def _layer_tc(h, aggp, degp, Ws, Wn, bb, ln_g, ln_b, residual):
    full = lambda shape: pl.BlockSpec(shape, lambda i: tuple(0 for _ in shape))
    return pl.pallas_call(
        functools.partial(_layer_body, residual),
        grid=(N // _BN,),
        in_specs=[pl.BlockSpec((_BN, H), lambda i: (i, 0)),
                  pl.BlockSpec((NC, _BN, H), lambda i: (0, i, 0)),
                  pl.BlockSpec((NC, _BN, 1), lambda i: (0, i, 0)),
                  full((H, H)), full((H, H)), full((1, H)),
                  full((1, H)), full((1, H))],
        out_specs=pl.BlockSpec((_BN, H), lambda i: (i, 0)),
        out_shape=jax.ShapeDtypeStruct((N, H), jnp.float32),
    )(h, aggp, degp.reshape(NC, N, 1), Ws, Wn, bb.reshape(1, H),
      ln_g.reshape(1, H), ln_b.reshape(1, H))


def kernel(x, edge_index_l0, edge_index_l1, pos_edge_index, neg_edge_index,
           W_in, b_in, lora_A, lora_B, pn_g, pn_b,
           Ws0, Wn0, bb0, ln0_g, ln0_b,
           Ws1, Wn1, bb1, ln1_g, ln1_b):
    h0 = _proj_tc(x, W_in, b_in, lora_A, lora_B, pn_g, pn_b)
    aggp0, degp0 = _sc_aggregate(h0, edge_index_l0[0], edge_index_l0[1])
    h1 = _layer_tc(h0, aggp0, degp0.reshape(NC, N, 1),
                   Ws0, Wn0, bb0, ln0_g, ln0_b, residual=False)
    aggp1, degp1 = _sc_aggregate(h1, edge_index_l1[0], edge_index_l1[1])
    h2 = _layer_tc(h1, aggp1, degp1.reshape(NC, N, 1),
                   Ws1, Wn1, bb1, ln1_g, ln1_b, residual=True)
    comb = jnp.concatenate(
        [pos_edge_index.reshape(-1), neg_edge_index.reshape(-1)])
    prows = _sc_scores(h2, comb)
    scores = _dots_tc(prows)[:, 0]
    return (scores[:EP], scores[EP:])


# speedup vs baseline: 1.3685x; 1.0347x over previous
import functools

import jax
import jax.numpy as jnp
from jax import lax
from jax.experimental import pallas as pl
from jax.experimental.pallas import tpu as pltpu
from jax.experimental.pallas import tpu_sc as plsc

N = 10000
D = 128
H = 128
E = 320000
EP = 10000

NC = 2
NS = 16
NW = NC * NS


KA = 80
EDGES_PER_TILE = E // NW
NGROUPS = EDGES_PER_TILE // KA
NWCH = N // KA
NRING = 4


def _sc_aggregate_body(h_hbm, src_hbm, dst_hbm, agg_out, deg_out,
                       src0, dst0, src1, dst1, src2, dst2, src3, dst3,
                       ones_v, rows0, rows1, rows2, rows3, zdeg_v,
                       agg_sh, deg_sh,
                       sem_g, sem_s, sem_w, sem_i0, sem_i1, sem_i2, sem_i3):
    c = lax.axis_index("c")
    s = lax.axis_index("s")
    rows = [rows0, rows1, rows2, rows3]
    isrc = [src0, src1, src2, src3]
    idst = [dst0, dst1, dst2, dst3]
    isem = [sem_i0, sem_i1, sem_i2, sem_i3]

    gbase = (c * NS + s) * NGROUPS
    pltpu.async_copy(src_hbm.at[gbase], isrc[0], isem[0])
    pltpu.async_copy(dst_hbm.at[gbase], idst[0], isem[0])
    pltpu.async_copy(src_hbm.at[gbase + 1], isrc[1], isem[1])
    pltpu.async_copy(dst_hbm.at[gbase + 1], idst[1], isem[1])

    zero16 = jnp.zeros((16,), jnp.float32)
    one16 = jnp.ones((16,), jnp.float32)
    for j in range(KA // 16):
        ones_v[pl.ds(j * 16, 16)] = one16
    for r in range(KA):
        for q in range(H // 16):
            rows3[r, pl.ds(q * 16, 16)] = zero16
    for j in range(2000 // 16):
        zdeg_v[pl.ds(j * 16, 16)] = zero16

    nz = (NWCH + NS - 1) // NS

    def zfire(k, carry):
        cid = s + k * NS

        @pl.when(cid < NWCH)
        def _():
            pltpu.async_copy(rows3, agg_sh.at[pl.ds(cid * KA, KA)], sem_s)

        return carry

    def zdrain(k, carry):
        cid = s + k * NS

        @pl.when(cid < NWCH)
        def _():
            pltpu.make_async_copy(
                rows3, agg_sh.at[pl.ds(cid * KA, KA)], sem_s).wait()

        return carry

    lax.fori_loop(0, nz, zfire, 0)

    @pl.when(s < 5)
    def _zero_deg():
        pltpu.async_copy(zdeg_v, deg_sh.at[pl.ds(s * 2000, 2000)], sem_s)

    pltpu.make_async_copy(src_hbm.at[gbase], isrc[0], isem[0]).wait()
    pltpu.make_async_copy(dst_hbm.at[gbase], idst[0], isem[0]).wait()
    pltpu.async_copy(h_hbm.at[isrc[0].at[0]], rows[0], sem_g)

    lax.fori_loop(0, nz, zdrain, 0)

    @pl.when(s < 5)
    def _zero_deg_wait():
        pltpu.make_async_copy(
            zdeg_v, deg_sh.at[pl.ds(s * 2000, 2000)], sem_s).wait()

    plsc.subcore_barrier()

    def process(g, j, do_drain, do_prefetch, do_gather_next):
        nj = (j + 1) % NRING
        pj = (j + 2) % NRING
        if do_drain:
            pltpu.make_async_copy(
                rows[pj], agg_sh.at[idst[pj].at[0]], sem_s).wait()
            pltpu.make_async_copy(
                ones_v, deg_sh.at[idst[pj].at[0]], sem_s).wait()
        if do_prefetch:
            pltpu.async_copy(src_hbm.at[gbase + g + 2], isrc[pj], isem[pj])
            pltpu.async_copy(dst_hbm.at[gbase + g + 2], idst[pj], isem[pj])
        if do_gather_next:
            pltpu.make_async_copy(
                src_hbm.at[gbase + g + 1], isrc[nj], isem[nj]).wait()
            pltpu.make_async_copy(
                dst_hbm.at[gbase + g + 1], idst[nj], isem[nj]).wait()
            pltpu.async_copy(h_hbm.at[isrc[nj].at[0]], rows[nj], sem_g)
        pltpu.make_async_copy(h_hbm.at[isrc[j].at[0]], rows[j], sem_g).wait()
        pltpu.async_copy(rows[j], agg_sh.at[idst[j].at[0]], sem_s, add=True)
        pltpu.async_copy(ones_v, deg_sh.at[idst[j].at[0]], sem_s, add=True)

    process(0, 0, False, True, True)
    process(1, 1, False, True, True)

    def quad(p, carry):
        g = 2 + 4 * p
        process(g, 2, True, True, True)
        process(g + 1, 3, True, True, True)
        process(g + 2, 0, True, True, True)
        process(g + 3, 1, True, True, True)
        return carry

    lax.fori_loop(0, (NGROUPS - 5) // 4, quad, 0)
    process(NGROUPS - 3, 2, True, True, True)
    process(NGROUPS - 2, 3, True, False, True)
    process(NGROUPS - 1, 0, True, False, False)

    for j in (3, 0):
        pltpu.make_async_copy(
            rows[j], agg_sh.at[idst[j].at[0]], sem_s).wait()
        pltpu.make_async_copy(
            ones_v, deg_sh.at[idst[j].at[0]], sem_s).wait()

    plsc.subcore_barrier()

    nw_rounds = (NWCH + NS * NRING - 1) // (NS * NRING)

    def wround(r, carry):
        for b in range(NRING):
            cid = s + (r * NRING + b) * NS

            @pl.when(cid < NWCH)
            def _():
                pltpu.async_copy(agg_sh.at[pl.ds(cid * KA, KA)],
                                 rows[b], sem_g)

        for b in range(NRING):
            cid = s + (r * NRING + b) * NS

            @pl.when(cid < NWCH)
            def _():
                pltpu.make_async_copy(agg_sh.at[pl.ds(cid * KA, KA)],
                                      rows[b], sem_g).wait()
                pltpu.async_copy(rows[b],
                                 agg_out.at[c, pl.ds(cid * KA, KA)],
                                 sem_w)

        for b in range(NRING):
            cid = s + (r * NRING + b) * NS

            @pl.when(cid < NWCH)
            def _():
                pltpu.make_async_copy(
                    rows[b], agg_out.at[c, pl.ds(cid * KA, KA)],
                    sem_w).wait()

        return carry

    lax.fori_loop(0, nw_rounds, wround, 0)

    @pl.when(s < 5)
    def _write_deg():
        pltpu.sync_copy(deg_sh.at[pl.ds(s * 2000, 2000)], zdeg_v)
        pltpu.sync_copy(zdeg_v, deg_out.at[pl.ds(c * N + s * 2000, 2000)])


def _sc_aggregate(h, src_arr, dst_arr):
    mesh = plsc.VectorSubcoreMesh(core_axis_name="c", subcore_axis_name="s")
    src3 = src_arr.reshape(NW * NGROUPS, 1, KA)
    dst3 = dst_arr.reshape(NW * NGROUPS, 1, KA)
    return pl.kernel(
        _sc_aggregate_body,
        out_type=[jax.ShapeDtypeStruct((NC, N, H), jnp.float32),
                  jax.ShapeDtypeStruct((NC * N,), jnp.float32)],
        mesh=mesh,
        scratch_types=[
            pltpu.VMEM((1, KA), jnp.int32),
            pltpu.VMEM((1, KA), jnp.int32),
            pltpu.VMEM((1, KA), jnp.int32),
            pltpu.VMEM((1, KA), jnp.int32),
            pltpu.VMEM((1, KA), jnp.int32),
            pltpu.VMEM((1, KA), jnp.int32),
            pltpu.VMEM((1, KA), jnp.int32),
            pltpu.VMEM((1, KA), jnp.int32),
            pltpu.VMEM((KA,), jnp.float32),
            pltpu.VMEM((KA, H), jnp.float32),
            pltpu.VMEM((KA, H), jnp.float32),
            pltpu.VMEM((KA, H), jnp.float32),
            pltpu.VMEM((KA, H), jnp.float32),
            pltpu.VMEM((2000,), jnp.float32),
            pltpu.VMEM_SHARED((N, H), jnp.float32),
            pltpu.VMEM_SHARED((N,), jnp.float32),
            pltpu.SemaphoreType.DMA,
            pltpu.SemaphoreType.DMA,
            pltpu.SemaphoreType.DMA,
            pltpu.SemaphoreType.DMA,
            pltpu.SemaphoreType.DMA,
            pltpu.SemaphoreType.DMA,
            pltpu.SemaphoreType.DMA,
        ],
    )(h, src3, dst3)


K = 80
NCHUNKS_SCORE = 2 * EP // K
CHUNKS_PER_ARRAY = EP // K
KMAX = (NCHUNKS_SCORE + NW - 1) // NW


def _sc_scores_body(h_hbm, comb_hbm, prows_out,
                    aA, bA, aB, bB, rows_aA, rows_bA, rows_aB, rows_bB,
                    semA, semB, siA, siB, so):
    c = lax.axis_index("c")
    s = lax.axis_index("s")
    w = s * NC + c

    aidx = [aA, aB]
    bidx = [bA, bB]
    rows_a = [rows_aA, rows_aB]
    rows_b = [rows_bA, rows_bB]
    sem_g = [semA, semB]
    sem_i = [siA, siB]

    def offs(k):
        cid = w + k * NW
        g = cid // CHUNKS_PER_ARRAY
        off = (cid % CHUNKS_PER_ARRAY) * K
        return cid, g, off

    def load_idx(k, x):
        cid, g, off = offs(k)

        @pl.when(cid < NCHUNKS_SCORE)
        def _():
            pltpu.async_copy(comb_hbm.at[pl.ds(2 * g * EP + off, K)],
                             aidx[x], sem_i[x])
            pltpu.async_copy(comb_hbm.at[pl.ds((2 * g + 1) * EP + off, K)],
                             bidx[x], sem_i[x])

    def issue_gathers(k, x):
        cid, g, off = offs(k)

        @pl.when(cid < NCHUNKS_SCORE)
        def _():
            pltpu.make_async_copy(
                comb_hbm.at[pl.ds(2 * g * EP + off, K)], aidx[x],
                sem_i[x]).wait()
            pltpu.make_async_copy(
                comb_hbm.at[pl.ds((2 * g + 1) * EP + off, K)], bidx[x],
                sem_i[x]).wait()
            pltpu.async_copy(h_hbm.at[aidx[x]], rows_a[x], sem_g[x])
            pltpu.async_copy(h_hbm.at[bidx[x]], rows_b[x], sem_g[x])

    def drain_out(k, x):
        cid, g, off = offs(k)

        @pl.when(cid < NCHUNKS_SCORE)
        def _():
            pltpu.make_async_copy(
                rows_a[x], prows_out.at[pl.ds(g * EP + off, K)], so).wait()

    def wait_gathers(k, x):
        cid, g, off = offs(k)

        @pl.when(cid < NCHUNKS_SCORE)
        def _():
            pltpu.make_async_copy(h_hbm.at[aidx[x]], rows_a[x],
                                  sem_g[x]).wait()
            pltpu.make_async_copy(h_hbm.at[bidx[x]], rows_b[x],
                                  sem_g[x]).wait()

    def compute_and_out(k, x):
        cid, g, off = offs(k)

        @pl.when(cid < NCHUNKS_SCORE)
        def _():
            def ebody(e, carry):
                for q in range(H // 16):
                    sl = pl.ds(q * 16, 16)
                    rows_a[x][e, sl] = rows_a[x][e, sl] * rows_b[x][e, sl]
                return carry

            lax.fori_loop(0, K, ebody, 0)
            pltpu.async_copy(rows_a[x], prows_out.at[pl.ds(g * EP + off, K)],
                             so)

    load_idx(0, 0)
    load_idx(1, 1)
    issue_gathers(0, 0)
    for k in range(KMAX):
        x = k % 2
        wait_gathers(k, x)
        if k + 2 < KMAX:
            load_idx(k + 2, x)
        if k >= 1:
            drain_out(k - 1, 1 - x)
        if k + 1 < KMAX:
            issue_gathers(k + 1, 1 - x)
        compute_and_out(k, x)
    drain_out(KMAX - 1, (KMAX - 1) % 2)


def _sc_scores(h, comb):
    mesh = plsc.VectorSubcoreMesh(core_axis_name="c", subcore_axis_name="s")
    return pl.kernel(
        _sc_scores_body,
        out_type=jax.ShapeDtypeStruct((2 * EP, H), jnp.float32),
        mesh=mesh,
        scratch_types=[
            pltpu.VMEM((K,), jnp.int32),
            pltpu.VMEM((K,), jnp.int32),
            pltpu.VMEM((K,), jnp.int32),
            pltpu.VMEM((K,), jnp.int32),
            pltpu.VMEM((K, H), jnp.float32),
            pltpu.VMEM((K, H), jnp.float32),
            pltpu.VMEM((K, H), jnp.float32),
            pltpu.VMEM((K, H), jnp.float32),
            pltpu.SemaphoreType.DMA,
            pltpu.SemaphoreType.DMA,
            pltpu.SemaphoreType.DMA,
            pltpu.SemaphoreType.DMA,
            pltpu.SemaphoreType.DMA,
        ],
    )(h, comb)


def _dots_body(p_ref, out_ref):
    out_ref[...] = jnp.sum(p_ref[...], axis=-1, keepdims=True)


def _dots_tc(prows):
    bn = 2000
    return pl.pallas_call(
        _dots_body,
        grid=(2 * EP // bn,),
        in_specs=[pl.BlockSpec((bn, H), lambda i: (i, 0))],
        out_specs=pl.BlockSpec((bn, 1), lambda i: (i, 0)),
        out_shape=jax.ShapeDtypeStruct((2 * EP, 1), jnp.float32),
    )(prows)


def _ln(t, g, b):
    m = jnp.mean(t, axis=-1, keepdims=True)
    v = jnp.mean((t - m) * (t - m), axis=-1, keepdims=True)
    return (t - m) * lax.rsqrt(v + 1e-5) * g + b


def _proj_body(x_ref, Win_ref, bin_ref, lA_ref, lB_ref, g_ref, b_ref, out_ref):
    x = x_ref[...]
    base = jnp.dot(x, Win_ref[...], preferred_element_type=jnp.float32)
    ada = jnp.dot(jnp.dot(x, lA_ref[...], preferred_element_type=jnp.float32),
                  lB_ref[...], preferred_element_type=jnp.float32)
    t = base + ada + bin_ref[...]
    out_ref[...] = _ln(t, g_ref[...], b_ref[...])


_BN = 1000


def _proj_tc(x, W_in, b_in, lora_A, lora_B, pn_g, pn_b):
    full = lambda shape: pl.BlockSpec(shape, lambda i: tuple(0 for _ in shape))
    return pl.pallas_call(
        _proj_body,
        grid=(N // _BN,),
        in_specs=[pl.BlockSpec((_BN, D), lambda i: (i, 0)),
                  full((D, H)), full((1, H)), full((D, 16)), full((16, H)),
                  full((1, H)), full((1, H))],
        out_specs=pl.BlockSpec((_BN, H), lambda i: (i, 0)),
        out_shape=jax.ShapeDtypeStruct((N, H), jnp.float32),
    )(x, W_in, b_in.reshape(1, H), lora_A, lora_B,
      pn_g.reshape(1, H), pn_b.reshape(1, H))


def _layer_body(residual, h_ref, agg_ref, deg_ref, Ws_ref, Wn_ref, bb_ref,
                g_ref, b_ref, out_ref):
    h = h_ref[...]
    agg = agg_ref[0] + agg_ref[1]
    deg = deg_ref[0] + deg_ref[1]
    agg = agg / jnp.maximum(deg, 1.0)
    t = (jnp.dot(h, Ws_ref[...], preferred_element_type=jnp.float32)
         + jnp.dot(agg, Wn_ref[...], preferred_element_type=jnp.float32)
         + bb_ref[...])
    t = jnp.maximum(t, 0.0)
    if residual:
        t = t + h
    out_ref[...] = _ln(t, g_ref[...], b_ref[...])


def _layer_tc(h, aggp, degp, Ws, Wn, bb, ln_g, ln_b, residual):
    full = lambda shape: pl.BlockSpec(shape, lambda i: tuple(0 for _ in shape))
    return pl.pallas_call(
        functools.partial(_layer_body, residual),
        grid=(N // _BN,),
        in_specs=[pl.BlockSpec((_BN, H), lambda i: (i, 0)),
                  pl.BlockSpec((NC, _BN, H), lambda i: (0, i, 0)),
                  pl.BlockSpec((NC, _BN, 1), lambda i: (0, i, 0)),
                  full((H, H)), full((H, H)), full((1, H)),
                  full((1, H)), full((1, H))],
        out_specs=pl.BlockSpec((_BN, H), lambda i: (i, 0)),
        out_shape=jax.ShapeDtypeStruct((N, H), jnp.float32),
    )(h, aggp, degp.reshape(NC, N, 1), Ws, Wn, bb.reshape(1, H),
      ln_g.reshape(1, H), ln_b.reshape(1, H))


def kernel(x, edge_index_l0, edge_index_l1, pos_edge_index, neg_edge_index,
           W_in, b_in, lora_A, lora_B, pn_g, pn_b,
           Ws0, Wn0, bb0, ln0_g, ln0_b,
           Ws1, Wn1, bb1, ln1_g, ln1_b):
    h0 = _proj_tc(x, W_in, b_in, lora_A, lora_B, pn_g, pn_b)
    aggp0, degp0 = _sc_aggregate(h0, edge_index_l0[0], edge_index_l0[1])
    h1 = _layer_tc(h0, aggp0, degp0.reshape(NC, N, 1),
                   Ws0, Wn0, bb0, ln0_g, ln0_b, residual=False)
    aggp1, degp1 = _sc_aggregate(h1, edge_index_l1[0], edge_index_l1[1])
    h2 = _layer_tc(h1, aggp1, degp1.reshape(NC, N, 1),
                   Ws1, Wn1, bb1, ln1_g, ln1_b, residual=True)
    comb = jnp.concatenate(
        [pos_edge_index.reshape(-1), neg_edge_index.reshape(-1)])
    prows = _sc_scores(h2, comb)
    scores = _dots_tc(prows)[:, 0]
    return (scores[:EP], scores[EP:])
